# baseline (speedup 1.0000x reference)
import dataclasses
import functools

import jax
import jax.numpy as jnp
from jax import lax
from jax.experimental import pallas as pl
from jax.experimental.pallas import tpu as pltpu
from jax.experimental.pallas import tpu_sc as plsc

_BR = 512
_R_SC = 2048
_RB = 4
_NC, _NS, _L = 2, 16, 16
_NW = _NC * _NS
_SEG = 4


def _tc_body(inv_ref, lab_ref, log_ref, out_ref):
    inv = inv_ref[0]
    logit = log_ref[:, :] * inv
    rm = jnp.max(logit, axis=1, keepdims=True)
    lse = jnp.log(jnp.sum(jnp.exp(logit - rm), axis=1)) + rm[:, 0]
    lab = lab_ref[:, :]
    m = jnp.max(lab, axis=1, keepdims=True)
    pos = jnp.max(jnp.where(lab == m, logit, -jnp.inf), axis=1)

    @pl.when(pl.program_id(0) == 0)
    def _init():
        out_ref[0, 0] = 0.0

    out_ref[0, 0] += jnp.sum(lse - pos)


def _combine_body(tc_ref, s_ref, w_ref, out_ref):
    out_ref[0, 0] = tc_ref[0, 0] + jnp.sum(jnp.log(s_ref[:, :]) + w_ref[:, :])


def _sc_body(lab_hbm, log_hbm, inv_hbm, s_hbm, w_hbm,
             labbuf, logbuf, invbuf, sbuf, wbuf,
             semA0, semB0, semA1, semB1, *, d, r_sc):
    r_s = r_sc // _NW
    nb = r_s // _RB
    nchunk = d // (_L * _SEG)
    wid = lax.axis_index("s") * _NC + lax.axis_index("c")
    row0 = wid * r_s
    g0 = wid * (r_s // _L)
    iota = lax.iota(jnp.int32, _L)
    neg_inf = jnp.full((_L,), -jnp.inf, jnp.float32)
    zeros = jnp.zeros((_L,), jnp.float32)
    izeros = jnp.zeros((_L,), jnp.int32)
    sems = ((semA0, semB0), (semA1, semB1))

    pltpu.async_copy(inv_hbm, invbuf, semA0).wait()
    invv = invbuf[...]

    def copies(b, slot):
        rbase = row0 + b * _RB
        return (pltpu.make_async_copy(lab_hbm.at[pl.ds(rbase, _RB)],
                                      labbuf.at[slot], sems[slot][0]),
                pltpu.make_async_copy(log_hbm.at[pl.ds(rbase, _RB)],
                                      logbuf.at[slot], sems[slot][1]))

    def start(b, slot):
        ca, cb = copies(b, slot)
        ca.start()
        cb.start()

    def wait(b, slot):
        ca, cb = copies(b, slot)
        ca.wait()
        cb.wait()

    def process(slot, svec, wvec, roff):
        lb = labbuf.at[slot]
        lg = logbuf.at[slot]
        for r in range(_RB):
            def p1(k, carry):
                kv = jnp.full((_L,), k, jnp.int32)
                mx = list(carry[:_SEG])
                ix = list(carry[_SEG:])
                for s in range(_SEG):
                    c = lb[r, pl.ds((s * nchunk + k) * _L, _L)]
                    gt = c > mx[s]
                    mx[s] = jnp.where(gt, c, mx[s])
                    ix[s] = jnp.where(gt, kv, ix[s])
                return tuple(mx) + tuple(ix)

            pp = lax.fori_loop(0, nchunk, p1,
                               (neg_inf,) * _SEG + (izeros,) * _SEG,
                               unroll=2)
            mx, ix = pp[0], pp[1 * _SEG]
            for s in range(1, _SEG):
                cand = pp[_SEG + s] + (s * nchunk)
                gt = pp[s] > mx
                mx = jnp.where(gt, pp[s], mx)
                ix = jnp.where(gt, cand, ix)
            m = jnp.max(mx)
            col = jnp.min(jnp.where(mx == m, ix * _L + iota, d))
            posv = plsc.load_gather(
                lg, [jnp.full((_L,), r, jnp.int32),
                     jnp.full((_L,), col, jnp.int32)])

            def p3(k, carry):
                out = []
                for s in range(_SEG):
                    cg = lg[r, pl.ds((s * nchunk + k) * _L, _L)]
                    out.append(carry[s] + jnp.exp(cg * invv))
                return tuple(out)

            ss = lax.fori_loop(0, nchunk, p3, (zeros,) * _SEG, unroll=2)
            s_row = jnp.sum((ss[0] + ss[1]) + (ss[2] + ss[3]))

            ridx = roff + r
            svec = jnp.where(iota == ridx, s_row, svec)
            wvec = jnp.where(iota == ridx, -posv * invv, wvec)
        return svec, wvec

    start(0, 0)
    start(1, 1)

    @pl.loop(0, nb, step=4)
    def _quad(b):
        svec, wvec = zeros, zeros
        for j in range(4):
            slot = j % 2
            wait(b + j, slot)
            svec, wvec = process(slot, svec, wvec, j * _RB)

            @pl.when(b + j + 2 < nb)
            def _next():
                start(b + j + 2, slot)

        sbuf[...] = svec
        wbuf[...] = wvec
        grow = g0 + b // 4
        cps = pltpu.make_async_copy(sbuf, s_hbm.at[grow], semA0)
        cpw = pltpu.make_async_copy(wbuf, w_hbm.at[grow], semB0)
        cps.start()
        cpw.start()
        cps.wait()
        cpw.wait()


@functools.partial(jax.jit, static_argnames=())
def kernel(labels, logits, mask, alpha):
    del mask
    n, d = logits.shape
    inv = (1.0 / alpha) * jnp.ones((1,), dtype=jnp.float32)
    invv = (1.0 / alpha) * jnp.ones((_L,), dtype=jnp.float32)
    r_off = _R_SC // _BR

    tc_out = pl.pallas_call(
        _tc_body,
        grid=((n - _R_SC) // _BR,),
        in_specs=[
            pl.BlockSpec(memory_space=pltpu.SMEM),
            pl.BlockSpec((_BR, d), lambda i: (r_off + i, 0)),
            pl.BlockSpec((_BR, d), lambda i: (r_off + i, 0)),
        ],
        out_specs=pl.BlockSpec(memory_space=pltpu.SMEM),
        out_shape=jax.ShapeDtypeStruct((1, 1), jnp.float32),
    )(inv, labels, logits)

    mesh = plsc.VectorSubcoreMesh(core_axis_name="c", subcore_axis_name="s")
    cp = pltpu.CompilerParams()
    if "needs_layout_passes" in pltpu.CompilerParams.__dataclass_fields__:
        cp = dataclasses.replace(cp, needs_layout_passes=False)
    sc_kernel = pl.kernel(
        functools.partial(_sc_body, d=d, r_sc=_R_SC),
        out_type=(jax.ShapeDtypeStruct((_R_SC // _L, _L), jnp.float32),
                  jax.ShapeDtypeStruct((_R_SC // _L, _L), jnp.float32)),
        mesh=mesh,
        scratch_types=[
            pltpu.VMEM((2, _RB, d), jnp.float32),
            pltpu.VMEM((2, _RB, d), jnp.float32),
            pltpu.VMEM((_L,), jnp.float32),
            pltpu.VMEM((_L,), jnp.float32),
            pltpu.VMEM((_L,), jnp.float32),
            pltpu.SemaphoreType.DMA,
            pltpu.SemaphoreType.DMA,
            pltpu.SemaphoreType.DMA,
            pltpu.SemaphoreType.DMA,
        ],
        compiler_params=cp,
    )
    s_sc, w_sc = sc_kernel(labels, logits, invv)

    out = pl.pallas_call(
        _combine_body,
        in_specs=[
            pl.BlockSpec(memory_space=pltpu.SMEM),
            pl.BlockSpec((_R_SC // _L, _L), lambda: (0, 0)),
            pl.BlockSpec((_R_SC // _L, _L), lambda: (0, 0)),
        ],
        out_specs=pl.BlockSpec(memory_space=pltpu.SMEM),
        out_shape=jax.ShapeDtypeStruct((1, 1), jnp.float32),
    )(tc_out, s_sc, w_sc)
    return out[0, 0] / n


# speedup vs baseline: 1.1241x; 1.1241x over previous
import functools

import jax
import jax.numpy as jnp
from jax.experimental import pallas as pl
from jax.experimental.pallas import tpu as pltpu

_BR = 512


def _nce_body(inv_ref, lab_ref, log_ref, out_ref):
    inv = inv_ref[0]
    lab = lab_ref[:, :]
    logit = log_ref[:, :] * inv
    m = jnp.max(lab, axis=1, keepdims=True)
    pos = jnp.max(jnp.where(lab == m, logit, -jnp.inf), axis=1)
    rm = jnp.max(logit, axis=1, keepdims=True)
    lse = jnp.log(jnp.sum(jnp.exp(logit - rm), axis=1)) + rm[:, 0]
    block_sum = jnp.sum(lse - pos)

    @pl.when(pl.program_id(0) == 0)
    def _init():
        out_ref[0, 0] = 0.0

    out_ref[0, 0] += block_sum


@functools.partial(jax.jit, static_argnames=())
def kernel(labels, logits, mask, alpha):
    del mask
    n, d = logits.shape
    inv = (1.0 / alpha) * jnp.ones((1,), dtype=jnp.float32)
    grid = n // _BR
    out = pl.pallas_call(
        _nce_body,
        grid=(grid,),
        in_specs=[
            pl.BlockSpec(memory_space=pltpu.SMEM),
            pl.BlockSpec((_BR, d), lambda i: (i, 0)),
            pl.BlockSpec((_BR, d), lambda i: (i, 0)),
        ],
        out_specs=pl.BlockSpec(memory_space=pltpu.SMEM),
        out_shape=jax.ShapeDtypeStruct((1, 1), jnp.float32),
    )(inv, labels, logits)
    return out[0, 0] / n


# speedup vs baseline: 1.1255x; 1.0012x over previous
import functools

import jax
import jax.numpy as jnp
from jax.experimental import pallas as pl
from jax.experimental.pallas import tpu as pltpu

_BR = 512


def _nce_body(inv_ref, lab_ref, log_ref, out_ref):
    inv = inv_ref[0]
    lab = lab_ref[:, :]
    lo = log_ref[:, :]
    m = jnp.max(lab, axis=1, keepdims=True)
    pos = jnp.max(jnp.where(lab == m, lo, -jnp.inf), axis=1)
    rm = jnp.max(lo, axis=1, keepdims=True)
    se = jnp.sum(jnp.exp((lo - rm) * inv), axis=1)
    block_sum = jnp.sum(jnp.log(se) + (rm[:, 0] - pos) * inv)

    @pl.when(pl.program_id(0) == 0)
    def _init():
        out_ref[0, 0] = 0.0

    out_ref[0, 0] += block_sum


@functools.partial(jax.jit, static_argnames=())
def kernel(labels, logits, mask, alpha):
    del mask
    n, d = logits.shape
    inv = (1.0 / alpha) * jnp.ones((1,), dtype=jnp.float32)
    grid = n // _BR
    out = pl.pallas_call(
        _nce_body,
        grid=(grid,),
        in_specs=[
            pl.BlockSpec(memory_space=pltpu.SMEM),
            pl.BlockSpec((_BR, d), lambda i: (i, 0)),
            pl.BlockSpec((_BR, d), lambda i: (i, 0)),
        ],
        out_specs=pl.BlockSpec(memory_space=pltpu.SMEM),
        out_shape=jax.ShapeDtypeStruct((1, 1), jnp.float32),
    )(inv, labels, logits)
    return out[0, 0] / n
